# Initial kernel scaffold; baseline (speedup 1.0000x reference)
import jax
import jax.numpy as jnp
from jax.experimental import pallas as pl


def kernel(xyz, batch, dense, hashtab):
    raise NotImplementedError("write your pallas kernel here")


# speedup vs baseline: 6.5268x; 6.5268x over previous
import functools

import numpy as np
import jax
import jax.numpy as jnp
from jax import lax
from jax.experimental import pallas as pl
from jax.experimental.pallas import tpu as pltpu
from jax.experimental.pallas import tpu_sc as plsc

_N_LEVELS = 16
_F = 4
_B_GROWTH = 1.38
_BASE_RES = 2
_N_ENTRIES = 262147
_PS = (1, 2654435761, 805459861)
_N_POINTS = 65536

_NUMS = np.array([int(_BASE_RES * _B_GROWTH ** i) for i in range(_N_LEVELS)],
                 dtype=np.int64)
_CNTS = _NUMS ** 3
_ES = (1.0 / (_NUMS - 1)).astype(np.float32)
_SH = _N_LEVELS
for _i in range(_N_LEVELS):
    if _CNTS[_i] > _N_ENTRIES:
        _SH = _i
        break
_CUM = np.cumsum(_CNTS)
_DENSE_ROWS = int(_CUM[_SH - 1])
_NH = _N_LEVELS - _SH
_TOTAL_ROWS = _DENSE_ROWS + _NH * _N_ENTRIES
_BASES = [0] + [int(_CUM[i]) for i in range(_SH - 1)]
_HASH_BASES = [_DENSE_ROWS + i * _N_ENTRIES for i in range(_NH)]

_TPAD = ((_TOTAL_ROWS + 16383) // 16384) * 16384

_M = _N_ENTRIES
_P2_32 = 212995

_NW = 32
_CHUNK = _N_POINTS // _NW
_CROWS = _CHUNK // 128


def _rowsum_body(x_ref, o_ref):
    x = x_ref[...]
    r = lax.broadcasted_iota(jnp.int32, (128, 32), 0)
    c = lax.broadcasted_iota(jnp.int32, (128, 32), 1)
    g = jnp.where(r // 4 == c, jnp.float32(1.0), jnp.float32(0.0))
    o_ref[...] = lax.dot_general(x, g, (((1,), (0,)), ((), ())),
                                 preferred_element_type=jnp.float32,
                                 precision=lax.Precision.HIGHEST)


def _rowsum_call(tbl4):
    n = _TPAD // 32
    return pl.pallas_call(
        _rowsum_body,
        grid=(n // 512,),
        in_specs=[pl.BlockSpec((512, 128), lambda i: (i, jnp.int32(0)))],
        out_specs=pl.BlockSpec((512, 32), lambda i: (i, jnp.int32(0))),
        out_shape=jax.ShapeDtypeStruct((n, 32), jnp.float32),
    )(tbl4.reshape(n, 128))


def _mul64(v_i32, prime):
    vu = v_i32.astype(jnp.uint32)
    pa = vu * jnp.uint32(prime >> 16)
    pb = vu * jnp.uint32(prime & 0xFFFF)
    lo_a = pa << jnp.uint32(16)
    lo = lo_a + pb
    carry = (lo < lo_a).astype(jnp.uint32)
    hi = (pa >> jnp.uint32(16)) + carry
    return hi, lo


def _mod_m(hi, lo):
    r = (lo & jnp.uint32(0x3FFFF)).astype(jnp.int32)
    q = (lo >> jnp.uint32(18)).astype(jnp.int32)
    m1 = r - q * 3
    m1 = jnp.where(m1 < 0, m1 + _M, m1)
    t = hi.astype(jnp.int32) * _P2_32 + m1
    r2 = t & 0x3FFFF
    q2 = t >> 18
    m2 = r2 - q2 * 3
    return jnp.where(m2 < 0, m2 + _M, m2)


def _idxw_body(x_ref, y_ref, z_ref, idx_ref, w_ref):
    coords = (x_ref[...], y_ref[...], z_ref[...])
    for l in range(_N_LEVELS):
        num = int(_NUMS[l])
        es = jnp.float32(_ES[l])
        ic = []
        wts = []
        for d in range(3):
            f = coords[d] / es
            i0 = jnp.clip(f.astype(jnp.int32), 0, num - 1)
            i1 = jnp.clip((f + jnp.float32(1.0)).astype(jnp.int32), 0, num - 1)
            off = f - i0.astype(jnp.float32)
            ic.append((i0, i1))
            wts.append((jnp.float32(1.0) - off, off))
        if l >= _SH:
            hy = [_mul64(ic[1][t], _PS[1]) for t in range(2)]
            hz = [_mul64(ic[2][t], _PS[2]) for t in range(2)]
            xu = [ic[0][t].astype(jnp.uint32) for t in range(2)]
        for c in range(8):
            a, b, cz = (c >> 2) & 1, (c >> 1) & 1, c & 1
            if l < _SH:
                idx = (ic[0][a] * (num * num) + ic[1][b] * num + ic[2][cz]
                       + _BASES[l])
            else:
                lo = xu[a] ^ hy[b][1] ^ hz[cz][1]
                hi = hy[b][0] ^ hz[cz][0]
                idx = _mod_m(hi, lo) + _HASH_BASES[l - _SH]
            w = (wts[0][a] * wts[1][b]) * wts[2][cz]
            k = l * 8 + c
            idx_ref[k, :, :] = idx
            w_ref[k, :, :] = w


def _idxw_call(xs, ys, zs):
    blk = 64
    grid = (512 // blk,)
    cs = pl.BlockSpec((blk, 128), lambda i: (i, jnp.int32(0)))
    os = pl.BlockSpec((128, blk, 128), lambda i: (jnp.int32(0), i, jnp.int32(0)))
    return pl.pallas_call(
        _idxw_body,
        grid=grid,
        in_specs=[cs, cs, cs],
        out_specs=[os, os],
        out_shape=[jax.ShapeDtypeStruct((128, 512, 128), jnp.int32),
                   jax.ShapeDtypeStruct((128, 512, 128), jnp.float32)],
    )(xs, ys, zs)


def _sc_body(tbl, idx3, wts3, out, idx_v, w_v, val_v, out_v, sem):
    wid = (lax.axis_index("s").astype(jnp.int32) * jnp.int32(2)
           + lax.axis_index("c").astype(jnp.int32))
    row0 = wid * jnp.int32(_CROWS)

    for l in range(_N_LEVELS):
        src = (pl.ds(jnp.int32(l * 8), 8), pl.ds(row0, _CROWS))
        pltpu.sync_copy(idx3.at[src[0], src[1], :], idx_v)
        pltpu.sync_copy(wts3.at[src[0], src[1], :], w_v)

        def _issue(v, _):
            c = jax.lax.shift_right_logical(v, jnp.int32(4))
            j = jax.lax.bitwise_and(v, jnp.int32(_CROWS - 1))
            pltpu.async_copy(tbl.at[idx_v.at[c, j]], val_v.at[c, j], sem)
            return jnp.int32(0)

        lax.fori_loop(jnp.int32(0), jnp.int32(8 * _CROWS), _issue,
                      jnp.int32(0))
        pltpu.make_async_copy(wts3.at[src[0], src[1], :], val_v, sem).wait()

        def _acc(v, _):
            j = jax.lax.shift_right_logical(v, jnp.int32(3))
            m = jax.lax.shift_left(jax.lax.bitwise_and(v, jnp.int32(7)),
                                   jnp.int32(4))
            s = pl.ds(m, 16)
            a = w_v[0, j, s] * val_v[0, j, s]
            for c in range(1, 8):
                a = a + w_v[c, j, s] * val_v[c, j, s]
            out_v[j, s] = a
            return jnp.int32(0)

        lax.fori_loop(jnp.int32(0), jnp.int32(_CROWS * 8), _acc,
                      jnp.int32(0))
        pltpu.sync_copy(out_v, out.at[jnp.int32(l), pl.ds(row0, _CROWS), :])


def _sc_call(tbl, idx3, wts3):
    mesh = plsc.VectorSubcoreMesh(core_axis_name="c", subcore_axis_name="s")
    f = functools.partial(
        pl.kernel,
        mesh=mesh,
        out_type=jax.ShapeDtypeStruct((_N_LEVELS, 512, 128), jnp.float32),
        scratch_types=[
            pltpu.VMEM((8, _CROWS, 128), jnp.int32),
            pltpu.VMEM((8, _CROWS, 128), jnp.float32),
            pltpu.VMEM((8, _CROWS, 128), jnp.float32),
            pltpu.VMEM((_CROWS, 128), jnp.float32),
            pltpu.SemaphoreType.DMA,
        ],
    )(_sc_body)
    return f(tbl, idx3, wts3)


def kernel(xyz, batch, dense, hashtab):
    xyz = xyz.astype(jnp.float32)
    tbl4 = jnp.concatenate(
        [dense.astype(jnp.float32),
         hashtab.astype(jnp.float32).reshape(_NH * _N_ENTRIES, _F)], axis=0)
    tbl4 = jnp.pad(tbl4, ((0, _TPAD - _TOTAL_ROWS), (0, 0)))
    tbl = _rowsum_call(tbl4).reshape(_TPAD)

    xs = xyz[:, 0].reshape(512, 128)
    ys = xyz[:, 1].reshape(512, 128)
    zs = xyz[:, 2].reshape(512, 128)
    idx3, wts3 = _idxw_call(xs, ys, zs)

    lv = _sc_call(tbl, idx3, wts3)
    lv = lv.reshape(_N_LEVELS, _N_POINTS).T
    return jnp.concatenate([xyz, lv], axis=-1)


# speedup vs baseline: 21.7355x; 3.3302x over previous
import functools

import numpy as np
import jax
import jax.numpy as jnp
from jax import lax
from jax.experimental import pallas as pl
from jax.experimental.pallas import tpu as pltpu
from jax.experimental.pallas import tpu_sc as plsc

_N_LEVELS = 16
_F = 4
_B_GROWTH = 1.38
_BASE_RES = 2
_N_ENTRIES = 262147
_PS = (1, 2654435761, 805459861)
_N_POINTS = 65536

_NUMS = np.array([int(_BASE_RES * _B_GROWTH ** i) for i in range(_N_LEVELS)],
                 dtype=np.int64)
_CNTS = _NUMS ** 3
_ES = (1.0 / (_NUMS - 1)).astype(np.float32)
_SH = _N_LEVELS
for _i in range(_N_LEVELS):
    if _CNTS[_i] > _N_ENTRIES:
        _SH = _i
        break
_CUM = np.cumsum(_CNTS)
_DENSE_ROWS = int(_CUM[_SH - 1])
_NH = _N_LEVELS - _SH
_TOTAL_ROWS = _DENSE_ROWS + _NH * _N_ENTRIES
_BASES = [0] + [int(_CUM[i]) for i in range(_SH - 1)]
_HASH_BASES = [_DENSE_ROWS + i * _N_ENTRIES for i in range(_NH)]

_TPAD = ((_TOTAL_ROWS + 16383) // 16384) * 16384

_M = _N_ENTRIES
_P2_32 = 212995

_NSUB = 16
_TROWS = 512 // _NSUB
_CORE_SPLIT = 13
_SC1_OFF = 724088
_STCH = 8192
_STSPAN = 6 * _STCH
_SC_SLICE = 16 * _STSPAN + _STCH


def _rowsum_body(x_ref, o_ref):
    x = x_ref[...]
    r = lax.broadcasted_iota(jnp.int32, (128, 32), 0)
    c = lax.broadcasted_iota(jnp.int32, (128, 32), 1)
    g = jnp.where(r // 4 == c, jnp.float32(1.0), jnp.float32(0.0))
    o_ref[...] = lax.dot_general(x, g, (((1,), (0,)), ((), ())),
                                 preferred_element_type=jnp.float32,
                                 precision=lax.Precision.HIGHEST)


def _rowsum_call(tbl4):
    n = _TPAD // 32
    return pl.pallas_call(
        _rowsum_body,
        grid=(n // 512,),
        in_specs=[pl.BlockSpec((512, 128), lambda i: (i, jnp.int32(0)))],
        out_specs=pl.BlockSpec((512, 32), lambda i: (i, jnp.int32(0))),
        out_shape=jax.ShapeDtypeStruct((n, 32), jnp.float32),
    )(tbl4.reshape(n, 128))


def _mul64(v_i32, prime):
    vu = v_i32.astype(jnp.uint32)
    pa = vu * jnp.uint32(prime >> 16)
    pb = vu * jnp.uint32(prime & 0xFFFF)
    lo_a = pa << jnp.uint32(16)
    lo = lo_a + pb
    carry = (lo < lo_a).astype(jnp.uint32)
    hi = (pa >> jnp.uint32(16)) + carry
    return hi, lo


def _mod_m(hi, lo):
    r = (lo & jnp.uint32(0x3FFFF)).astype(jnp.int32)
    q = (lo >> jnp.uint32(18)).astype(jnp.int32)
    m1 = r - q * 3
    m1 = jnp.where(m1 < 0, m1 + _M, m1)
    t = hi.astype(jnp.int32) * _P2_32 + m1
    r2 = t & 0x3FFFF
    q2 = t >> 18
    m2 = r2 - q2 * 3
    return jnp.where(m2 < 0, m2 + _M, m2)


def _idxw_body(x_ref, y_ref, z_ref, idx_ref, w_ref):
    coords = (x_ref[...], y_ref[...], z_ref[...])
    for l in range(_N_LEVELS):
        num = int(_NUMS[l])
        es = jnp.float32(_ES[l])
        ic = []
        wts = []
        for d in range(3):
            f = coords[d] / es
            i0 = jnp.clip(f.astype(jnp.int32), 0, num - 1)
            i1 = jnp.clip((f + jnp.float32(1.0)).astype(jnp.int32), 0, num - 1)
            off = f - i0.astype(jnp.float32)
            ic.append((i0, i1))
            wts.append((jnp.float32(1.0) - off, off))
        if l >= _SH:
            hy = [_mul64(ic[1][t], _PS[1]) for t in range(2)]
            hz = [_mul64(ic[2][t], _PS[2]) for t in range(2)]
            xu = [ic[0][t].astype(jnp.uint32) for t in range(2)]
        for c in range(8):
            a, b, cz = (c >> 2) & 1, (c >> 1) & 1, c & 1
            if l < _SH:
                idx = (ic[0][a] * (num * num) + ic[1][b] * num + ic[2][cz]
                       + _BASES[l])
            else:
                lo = xu[a] ^ hy[b][1] ^ hz[cz][1]
                hi = hy[b][0] ^ hz[cz][0]
                base = _HASH_BASES[l - _SH] - (_SC1_OFF if l >= _CORE_SPLIT
                                               else 0)
                idx = _mod_m(hi, lo) + base
            w = (wts[0][a] * wts[1][b]) * wts[2][cz]
            k = l * 8 + c
            idx_ref[k, :, :] = idx
            w_ref[k, :, :] = w


def _idxw_call(xs, ys, zs):
    blk = 64
    grid = (512 // blk,)
    cs = pl.BlockSpec((blk, 128), lambda i: (i, jnp.int32(0)))
    os = pl.BlockSpec((128, blk, 128), lambda i: (jnp.int32(0), i, jnp.int32(0)))
    return pl.pallas_call(
        _idxw_body,
        grid=grid,
        in_specs=[cs, cs, cs],
        out_specs=[os, os],
        out_shape=[jax.ShapeDtypeStruct((128, 512, 128), jnp.int32),
                   jax.ShapeDtypeStruct((128, 512, 128), jnp.float32)],
    )(xs, ys, zs)


def _sc_body(tbl, idx3, wts3, out, idx_v, w_v, val_v, out_v, stage_v, shared,
             sem):
    sid = lax.axis_index("s").astype(jnp.int32)
    cid = lax.axis_index("c").astype(jnp.int32)
    row0 = sid * jnp.int32(_TROWS)

    coreoff = cid * jnp.int32(_SC1_OFF)
    myoff = sid * jnp.int32(_STSPAN)

    def _stage_chunk(i, _):
        off = myoff + i * jnp.int32(_STCH)
        pltpu.sync_copy(tbl.at[pl.ds(coreoff + off, _STCH)], stage_v)
        pltpu.sync_copy(stage_v, shared.at[pl.ds(off, _STCH)])
        return jnp.int32(0)

    nch = jnp.where(sid == jnp.int32(_NSUB - 1), jnp.int32(7), jnp.int32(6))
    lax.fori_loop(jnp.int32(0), nch, _stage_chunk, jnp.int32(0))
    plsc.subcore_barrier()

    for l in range(_N_LEVELS):
        lvl_core = 0 if l < _CORE_SPLIT else 1

        @pl.when(cid == jnp.int32(lvl_core))
        def _level(l=l):
            for h in range(2):
                src = (pl.ds(jnp.int32(l * 8 + h * 4), 4),
                       pl.ds(row0, _TROWS))
                pltpu.sync_copy(idx3.at[src[0], src[1], :], idx_v)
                pltpu.sync_copy(wts3.at[src[0], src[1], :], w_v)

                def _issue(v, _):
                    c = jax.lax.shift_right_logical(v, jnp.int32(5))
                    j = jax.lax.bitwise_and(v, jnp.int32(_TROWS - 1))
                    pltpu.async_copy(shared.at[idx_v.at[c, j]],
                                     val_v.at[c, j], sem)
                    return jnp.int32(0)

                lax.fori_loop(jnp.int32(0), jnp.int32(4 * _TROWS), _issue,
                              jnp.int32(0))
                pltpu.make_async_copy(wts3.at[src[0], src[1], :], val_v,
                                      sem).wait()

                def _acc(v, _):
                    j = jax.lax.shift_right_logical(v, jnp.int32(3))
                    m = jax.lax.shift_left(
                        jax.lax.bitwise_and(v, jnp.int32(7)), jnp.int32(4))
                    s = pl.ds(m, 16)
                    a = w_v[0, j, s] * val_v[0, j, s]
                    for c in range(1, 4):
                        a = a + w_v[c, j, s] * val_v[c, j, s]
                    if h == 1:
                        a = out_v[j, s] + a
                    out_v[j, s] = a
                    return jnp.int32(0)

                lax.fori_loop(jnp.int32(0), jnp.int32(_TROWS * 8), _acc,
                              jnp.int32(0))
            pltpu.sync_copy(out_v,
                            out.at[jnp.int32(l), pl.ds(row0, _TROWS), :])


def _sc_call(tbl, idx3, wts3):
    mesh = plsc.VectorSubcoreMesh(core_axis_name="c", subcore_axis_name="s")
    f = functools.partial(
        pl.kernel,
        mesh=mesh,
        out_type=jax.ShapeDtypeStruct((_N_LEVELS, 512, 128), jnp.float32),
        scratch_types=[
            pltpu.VMEM((4, _TROWS, 128), jnp.int32),
            pltpu.VMEM((4, _TROWS, 128), jnp.float32),
            pltpu.VMEM((4, _TROWS, 128), jnp.float32),
            pltpu.VMEM((_TROWS, 128), jnp.float32),
            pltpu.VMEM((_STCH,), jnp.float32),
            pltpu.VMEM_SHARED((_SC_SLICE,), jnp.float32),
            pltpu.SemaphoreType.DMA,
        ],
    )(_sc_body)
    return f(tbl, idx3, wts3)


def kernel(xyz, batch, dense, hashtab):
    xyz = xyz.astype(jnp.float32)
    tbl4 = jnp.concatenate(
        [dense.astype(jnp.float32),
         hashtab.astype(jnp.float32).reshape(_NH * _N_ENTRIES, _F)], axis=0)
    tbl4 = jnp.pad(tbl4, ((0, _TPAD - _TOTAL_ROWS), (0, 0)))
    tbl = _rowsum_call(tbl4).reshape(_TPAD)

    xs = xyz[:, 0].reshape(512, 128)
    ys = xyz[:, 1].reshape(512, 128)
    zs = xyz[:, 2].reshape(512, 128)
    idx3, wts3 = _idxw_call(xs, ys, zs)

    lv = _sc_call(tbl, idx3, wts3)
    lv = lv.reshape(_N_LEVELS, _N_POINTS).T
    return jnp.concatenate([xyz, lv], axis=-1)


# speedup vs baseline: 49.1102x; 2.2594x over previous
import functools

import numpy as np
import jax
import jax.numpy as jnp
from jax import lax
from jax.experimental import pallas as pl
from jax.experimental.pallas import tpu as pltpu
from jax.experimental.pallas import tpu_sc as plsc

_N_LEVELS = 16
_F = 4
_B_GROWTH = 1.38
_BASE_RES = 2
_N_ENTRIES = 262147
_PS = (1, 2654435761, 805459861)
_N_POINTS = 65536

_NUMS = np.array([int(_BASE_RES * _B_GROWTH ** i) for i in range(_N_LEVELS)],
                 dtype=np.int64)
_CNTS = _NUMS ** 3
_ES = (1.0 / (_NUMS - 1)).astype(np.float32)
_SH = _N_LEVELS
for _i in range(_N_LEVELS):
    if _CNTS[_i] > _N_ENTRIES:
        _SH = _i
        break
_CUM = np.cumsum(_CNTS)
_DENSE_ROWS = int(_CUM[_SH - 1])
_NH = _N_LEVELS - _SH
_TOTAL_ROWS = _DENSE_ROWS + _NH * _N_ENTRIES
_BASES = [0] + [int(_CUM[i]) for i in range(_SH - 1)]
_HASH_BASES = [_DENSE_ROWS + i * _N_ENTRIES for i in range(_NH)]

_M = _N_ENTRIES
_P2_32 = 212995

_NSUB = 16
_TROWS = 512 // _NSUB
_CORE_SPLIT = 13
_STCH = 4096
_DROWS_PAD = 200704
_DCH = _DROWS_PAD // _STCH
_HSTRIDE = 278528
_HCH = _HSTRIDE // _STCH
_SC1_OFF = _DROWS_PAD + 2 * _HSTRIDE
_SC_SLICE = 3 * _HSTRIDE


def _rowsum_dense_body(x_ref, o_ref):
    o_ref[...] = jnp.sum(x_ref[...], axis=1)


def _rowsum_dense_call(dense):
    return pl.pallas_call(
        _rowsum_dense_body,
        grid=(_DROWS_PAD // 2048,),
        in_specs=[pl.BlockSpec((2048, _F), lambda i: (i, jnp.int32(0)))],
        out_specs=pl.BlockSpec((2048,), lambda i: (i,)),
        out_shape=jax.ShapeDtypeStruct((_DROWS_PAD,), jnp.float32),
    )(dense)


def _rowsum_hash_body(x_ref, o_ref):
    o_ref[...] = jnp.sum(x_ref[0], axis=1)


def _rowsum_hash_call(hashtab):
    nj = _HSTRIDE // 16384
    return pl.pallas_call(
        _rowsum_hash_body,
        grid=(_NH, nj),
        in_specs=[pl.BlockSpec((1, 16384, _F),
                               lambda i, j: (i, j, jnp.int32(0)))],
        out_specs=pl.BlockSpec((16384,), lambda i, j: (i * nj + j,)),
        out_shape=jax.ShapeDtypeStruct((_NH * _HSTRIDE,), jnp.float32),
    )(hashtab)


def _mul64(v_i32, prime):
    vu = v_i32.astype(jnp.uint32)
    pa = vu * jnp.uint32(prime >> 16)
    pb = vu * jnp.uint32(prime & 0xFFFF)
    lo_a = pa << jnp.uint32(16)
    lo = lo_a + pb
    carry = (lo < lo_a).astype(jnp.uint32)
    hi = (pa >> jnp.uint32(16)) + carry
    return hi, lo


def _mod_m(hi, lo):
    r = (lo & jnp.uint32(0x3FFFF)).astype(jnp.int32)
    q = (lo >> jnp.uint32(18)).astype(jnp.int32)
    m1 = r - q * 3
    m1 = jnp.where(m1 < 0, m1 + _M, m1)
    t = hi.astype(jnp.int32) * _P2_32 + m1
    r2 = t & 0x3FFFF
    q2 = t >> 18
    m2 = r2 - q2 * 3
    return jnp.where(m2 < 0, m2 + _M, m2)


def _idxw_body(x_ref, y_ref, z_ref, idx_ref, w_ref):
    coords = (x_ref[...], y_ref[...], z_ref[...])
    for l in range(_N_LEVELS):
        num = int(_NUMS[l])
        es = jnp.float32(_ES[l])
        ic = []
        wts = []
        for d in range(3):
            f = coords[d] / es
            i0 = jnp.clip(f.astype(jnp.int32), 0, num - 1)
            i1 = jnp.clip((f + jnp.float32(1.0)).astype(jnp.int32), 0, num - 1)
            off = f - i0.astype(jnp.float32)
            ic.append((i0, i1))
            wts.append((jnp.float32(1.0) - off, off))
        if l >= _SH:
            hy = [_mul64(ic[1][t], _PS[1]) for t in range(2)]
            hz = [_mul64(ic[2][t], _PS[2]) for t in range(2)]
            xu = [ic[0][t].astype(jnp.uint32) for t in range(2)]
        for c in range(8):
            a, b, cz = (c >> 2) & 1, (c >> 1) & 1, c & 1
            if l < _SH:
                idx = (ic[0][a] * (num * num) + ic[1][b] * num + ic[2][cz]
                       + _BASES[l])
            else:
                lo = xu[a] ^ hy[b][1] ^ hz[cz][1]
                hi = hy[b][0] ^ hz[cz][0]
                base = (_DROWS_PAD + (l - _SH) * _HSTRIDE
                        - (_SC1_OFF if l >= _CORE_SPLIT else 0))
                idx = _mod_m(hi, lo) + base
            w = (wts[0][a] * wts[1][b]) * wts[2][cz]
            k = l * 8 + c
            idx_ref[k, :, :] = idx
            w_ref[k, :, :] = w


def _idxw_call(xs, ys, zs):
    blk = 64
    grid = (512 // blk,)
    cs = pl.BlockSpec((blk, 128), lambda i: (i, jnp.int32(0)))
    os = pl.BlockSpec((128, blk, 128), lambda i: (jnp.int32(0), i, jnp.int32(0)))
    return pl.pallas_call(
        _idxw_body,
        grid=grid,
        in_specs=[cs, cs, cs],
        out_specs=[os, os],
        out_shape=[jax.ShapeDtypeStruct((128, 512, 128), jnp.int32),
                   jax.ShapeDtypeStruct((128, 512, 128), jnp.float32)],
    )(xs, ys, zs)


def _sc_body(dsum, hsum, idx3, wts3, out, idx_v, w_v, val_v, out_v, stage_v,
             shared, sem):
    sid = lax.axis_index("s").astype(jnp.int32)
    cid = lax.axis_index("c").astype(jnp.int32)
    row0 = sid * jnp.int32(_TROWS)

    def _stage(src_ref, src_base, nch, dst_base):
        def _chunk(i, _):
            k = sid + i * jnp.int32(_NSUB)

            @pl.when(k < jnp.int32(nch))
            def _do():
                off = k * jnp.int32(_STCH)
                pltpu.sync_copy(
                    src_ref.at[pl.ds(jnp.int32(src_base) + off, _STCH)],
                    stage_v)
                pltpu.sync_copy(
                    stage_v, shared.at[pl.ds(jnp.int32(dst_base) + off,
                                             _STCH)])

            return jnp.int32(0)

        lax.fori_loop(jnp.int32(0), jnp.int32(-(-nch // _NSUB)), _chunk,
                      jnp.int32(0))

    @pl.when(cid == jnp.int32(0))
    def _stage0():
        _stage(dsum, 0, _DCH, 0)
        _stage(hsum, 0, 2 * _HCH, _DROWS_PAD)

    @pl.when(cid == jnp.int32(1))
    def _stage1():
        _stage(hsum, 2 * _HSTRIDE, 3 * _HCH, 0)

    plsc.subcore_barrier()

    for l in range(_N_LEVELS):
        lvl_core = 0 if l < _CORE_SPLIT else 1

        @pl.when(cid == jnp.int32(lvl_core))
        def _level(l=l):
            for h in range(2):
                src = (pl.ds(jnp.int32(l * 8 + h * 4), 4),
                       pl.ds(row0, _TROWS))
                pltpu.sync_copy(idx3.at[src[0], src[1], :], idx_v)
                pltpu.sync_copy(wts3.at[src[0], src[1], :], w_v)

                def _issue(v, _):
                    c = jax.lax.shift_right_logical(v, jnp.int32(5))
                    j = jax.lax.bitwise_and(v, jnp.int32(_TROWS - 1))
                    pltpu.async_copy(shared.at[idx_v.at[c, j]],
                                     val_v.at[c, j], sem)
                    return jnp.int32(0)

                lax.fori_loop(jnp.int32(0), jnp.int32(4 * _TROWS), _issue,
                              jnp.int32(0))
                pltpu.make_async_copy(wts3.at[src[0], src[1], :], val_v,
                                      sem).wait()

                def _acc(v, _):
                    j = jax.lax.shift_right_logical(v, jnp.int32(3))
                    m = jax.lax.shift_left(
                        jax.lax.bitwise_and(v, jnp.int32(7)), jnp.int32(4))
                    s = pl.ds(m, 16)
                    a = w_v[0, j, s] * val_v[0, j, s]
                    for c in range(1, 4):
                        a = a + w_v[c, j, s] * val_v[c, j, s]
                    if h == 1:
                        a = out_v[j, s] + a
                    out_v[j, s] = a
                    return jnp.int32(0)

                lax.fori_loop(jnp.int32(0), jnp.int32(_TROWS * 8), _acc,
                              jnp.int32(0))
            pltpu.sync_copy(out_v,
                            out.at[jnp.int32(l), pl.ds(row0, _TROWS), :])


def _sc_call(dsum, hsum, idx3, wts3):
    mesh = plsc.VectorSubcoreMesh(core_axis_name="c", subcore_axis_name="s")
    f = functools.partial(
        pl.kernel,
        mesh=mesh,
        out_type=jax.ShapeDtypeStruct((_N_LEVELS, 512, 128), jnp.float32),
        scratch_types=[
            pltpu.VMEM((4, _TROWS, 128), jnp.int32),
            pltpu.VMEM((4, _TROWS, 128), jnp.float32),
            pltpu.VMEM((4, _TROWS, 128), jnp.float32),
            pltpu.VMEM((_TROWS, 128), jnp.float32),
            pltpu.VMEM((_STCH,), jnp.float32),
            pltpu.VMEM_SHARED((_SC_SLICE,), jnp.float32),
            pltpu.SemaphoreType.DMA,
        ],
    )(_sc_body)
    return f(dsum, hsum, idx3, wts3)


def kernel(xyz, batch, dense, hashtab):
    xyz = xyz.astype(jnp.float32)
    dsum = _rowsum_dense_call(dense.astype(jnp.float32))
    hsum = _rowsum_hash_call(hashtab.astype(jnp.float32))

    xs = xyz[:, 0].reshape(512, 128)
    ys = xyz[:, 1].reshape(512, 128)
    zs = xyz[:, 2].reshape(512, 128)
    idx3, wts3 = _idxw_call(xs, ys, zs)

    lv = _sc_call(dsum, hsum, idx3, wts3)
    lv = lv.reshape(_N_LEVELS, _N_POINTS).T
    return jnp.concatenate([xyz, lv], axis=-1)


# speedup vs baseline: 60.2946x; 1.2277x over previous
import functools

import numpy as np
import jax
import jax.numpy as jnp
from jax import lax
from jax.experimental import pallas as pl
from jax.experimental.pallas import tpu as pltpu
from jax.experimental.pallas import tpu_sc as plsc

_N_LEVELS = 16
_F = 4
_B_GROWTH = 1.38
_BASE_RES = 2
_N_ENTRIES = 262147
_PS = (1, 2654435761, 805459861)
_N_POINTS = 65536

_NUMS = np.array([int(_BASE_RES * _B_GROWTH ** i) for i in range(_N_LEVELS)],
                 dtype=np.int64)
_CNTS = _NUMS ** 3
_ES = (1.0 / (_NUMS - 1)).astype(np.float32)
_SH = _N_LEVELS
for _i in range(_N_LEVELS):
    if _CNTS[_i] > _N_ENTRIES:
        _SH = _i
        break
_CUM = np.cumsum(_CNTS)
_DENSE_ROWS = int(_CUM[_SH - 1])
_NH = _N_LEVELS - _SH
_TOTAL_ROWS = _DENSE_ROWS + _NH * _N_ENTRIES
_BASES = [0] + [int(_CUM[i]) for i in range(_SH - 1)]
_HASH_BASES = [_DENSE_ROWS + i * _N_ENTRIES for i in range(_NH)]

_M = _N_ENTRIES
_P2_32 = 212995

_NSUB = 16
_TROWS = 512 // _NSUB
_CORE_SPLIT = 13
_STCH = 4096
_DROWS_PAD = 200704
_DCH = _DROWS_PAD // _STCH
_HSTRIDE = 278528
_HCH = _HSTRIDE // _STCH
_SC1_OFF = 2 * _HSTRIDE
_SC_SLICE = _DROWS_PAD + 3 * _HSTRIDE


def _rowsum_dense_body(x_ref, o_ref):
    o_ref[...] = jnp.sum(x_ref[...], axis=1)


def _rowsum_dense_call(dense):
    return pl.pallas_call(
        _rowsum_dense_body,
        grid=(_DROWS_PAD // 2048,),
        in_specs=[pl.BlockSpec((2048, _F), lambda i: (i, jnp.int32(0)))],
        out_specs=pl.BlockSpec((2048,), lambda i: (i,)),
        out_shape=jax.ShapeDtypeStruct((_DROWS_PAD,), jnp.float32),
    )(dense)


def _rowsum_hash_body(x_ref, o_ref):
    o_ref[...] = jnp.sum(x_ref[0], axis=1)


def _rowsum_hash_call(hashtab):
    nj = _HSTRIDE // 16384
    return pl.pallas_call(
        _rowsum_hash_body,
        grid=(_NH, nj),
        in_specs=[pl.BlockSpec((1, 16384, _F),
                               lambda i, j: (i, j, jnp.int32(0)))],
        out_specs=pl.BlockSpec((16384,), lambda i, j: (i * nj + j,)),
        out_shape=jax.ShapeDtypeStruct((_NH * _HSTRIDE,), jnp.float32),
    )(hashtab)


def _mul64(v_i32, prime):
    vu = v_i32.astype(jnp.uint32)
    pa = vu * jnp.uint32(prime >> 16)
    pb = vu * jnp.uint32(prime & 0xFFFF)
    lo_a = pa << jnp.uint32(16)
    lo = lo_a + pb
    carry = (lo < lo_a).astype(jnp.uint32)
    hi = (pa >> jnp.uint32(16)) + carry
    return hi, lo


def _mod_m(hi, lo):
    r = (lo & jnp.uint32(0x3FFFF)).astype(jnp.int32)
    q = (lo >> jnp.uint32(18)).astype(jnp.int32)
    m1 = r - q * 3
    m1 = jnp.where(m1 < 0, m1 + _M, m1)
    t = hi.astype(jnp.int32) * _P2_32 + m1
    r2 = t & 0x3FFFF
    q2 = t >> 18
    m2 = r2 - q2 * 3
    return jnp.where(m2 < 0, m2 + _M, m2)


def _idxw_body(x_ref, y_ref, z_ref, idx_ref, w_ref):
    coords = (x_ref[...], y_ref[...], z_ref[...])
    for l in range(_N_LEVELS):
        num = int(_NUMS[l])
        es = jnp.float32(_ES[l])
        ic = []
        wts = []
        for d in range(3):
            f = coords[d] / es
            i0 = jnp.clip(f.astype(jnp.int32), 0, num - 1)
            i1 = jnp.clip((f + jnp.float32(1.0)).astype(jnp.int32), 0, num - 1)
            off = f - i0.astype(jnp.float32)
            ic.append((i0, i1))
            wts.append((jnp.float32(1.0) - off, off))
        if l >= _SH:
            hy = [_mul64(ic[1][t], _PS[1]) for t in range(2)]
            hz = [_mul64(ic[2][t], _PS[2]) for t in range(2)]
            xu = [ic[0][t].astype(jnp.uint32) for t in range(2)]
        for c in range(8):
            a, b, cz = (c >> 2) & 1, (c >> 1) & 1, c & 1
            if l < _SH:
                idx = (ic[0][a] * (num * num) + ic[1][b] * num + ic[2][cz]
                       + _BASES[l])
            else:
                lo = xu[a] ^ hy[b][1] ^ hz[cz][1]
                hi = hy[b][0] ^ hz[cz][0]
                base = (_DROWS_PAD + (l - _SH) * _HSTRIDE
                        - (_SC1_OFF if l >= _CORE_SPLIT else 0))
                idx = _mod_m(hi, lo) + base
            w = (wts[0][a] * wts[1][b]) * wts[2][cz]
            k = l * 8 + c
            idx_ref[k, :, :] = idx
            w_ref[k, :, :] = w


def _idxw_call(xs, ys, zs):
    blk = 64
    grid = (512 // blk,)
    cs = pl.BlockSpec((blk, 128), lambda i: (i, jnp.int32(0)))
    os = pl.BlockSpec((128, blk, 128), lambda i: (jnp.int32(0), i, jnp.int32(0)))
    return pl.pallas_call(
        _idxw_body,
        grid=grid,
        in_specs=[cs, cs, cs],
        out_specs=[os, os],
        out_shape=[jax.ShapeDtypeStruct((128, 512, 128), jnp.int32),
                   jax.ShapeDtypeStruct((128, 512, 128), jnp.float32)],
    )(xs, ys, zs)


def _sc_body(dsum, hsum, idx3, wts3, out, idx_v, w_v, val_v, out_v, stage_v,
             shared, sem):
    sid = lax.axis_index("s").astype(jnp.int32)
    cid = lax.axis_index("c").astype(jnp.int32)
    row0 = sid * jnp.int32(_TROWS)

    def _stage(src_ref, src_base, nch, dst_base):
        def _chunk(i, _):
            k = sid + i * jnp.int32(_NSUB)

            @pl.when(k < jnp.int32(nch))
            def _do():
                off = k * jnp.int32(_STCH)
                pltpu.sync_copy(
                    src_ref.at[pl.ds(jnp.int32(src_base) + off, _STCH)],
                    stage_v)
                pltpu.sync_copy(
                    stage_v, shared.at[pl.ds(jnp.int32(dst_base) + off,
                                             _STCH)])

            return jnp.int32(0)

        lax.fori_loop(jnp.int32(0), jnp.int32(-(-nch // _NSUB)), _chunk,
                      jnp.int32(0))

    _stage(dsum, 0, _DCH, 0)

    @pl.when(cid == jnp.int32(0))
    def _stage0():
        _stage(hsum, 0, 2 * _HCH, _DROWS_PAD)

    @pl.when(cid == jnp.int32(1))
    def _stage1():
        _stage(hsum, 2 * _HSTRIDE, 3 * _HCH, _DROWS_PAD)

    plsc.subcore_barrier()

    half = _TROWS // 2
    drow0 = cid * jnp.int32(512 // 2) + sid * jnp.int32(half)

    def _level(l, r0, nrows, jshift):
        for h in range(2):
            src = (pl.ds(jnp.int32(l * 8 + h * 4), 4), pl.ds(r0, nrows))
            dst = (slice(None), pl.ds(0, nrows), slice(None))
            pltpu.sync_copy(idx3.at[src[0], src[1], :], idx_v.at[dst])
            pltpu.sync_copy(wts3.at[src[0], src[1], :], w_v.at[dst])

            def _issue(v, _):
                c = jax.lax.shift_right_logical(v, jnp.int32(jshift))
                j = jax.lax.bitwise_and(v, jnp.int32(nrows - 1))
                pltpu.async_copy(shared.at[idx_v.at[c, j]], val_v.at[c, j],
                                 sem)
                return jnp.int32(0)

            lax.fori_loop(jnp.int32(0), jnp.int32(4 * nrows), _issue,
                          jnp.int32(0))
            pltpu.make_async_copy(wts3.at[src[0], src[1], :],
                                  val_v.at[dst], sem).wait()

            def _acc(v, _):
                j = jax.lax.shift_right_logical(v, jnp.int32(3))
                m = jax.lax.shift_left(
                    jax.lax.bitwise_and(v, jnp.int32(7)), jnp.int32(4))
                s = pl.ds(m, 16)
                a = w_v[0, j, s] * val_v[0, j, s]
                for c in range(1, 4):
                    a = a + w_v[c, j, s] * val_v[c, j, s]
                if h == 1:
                    a = out_v[j, s] + a
                out_v[j, s] = a
                return jnp.int32(0)

            lax.fori_loop(jnp.int32(0), jnp.int32(nrows * 8), _acc,
                          jnp.int32(0))
        pltpu.sync_copy(out_v.at[pl.ds(0, nrows), :],
                        out.at[jnp.int32(l), pl.ds(r0, nrows), :])

    for l in range(_N_LEVELS):
        if l < _SH:
            _level(l, drow0, half, 4)
        else:
            lvl_core = 0 if l < _CORE_SPLIT else 1

            @pl.when(cid == jnp.int32(lvl_core))
            def _hash_level(l=l):
                _level(l, row0, _TROWS, 5)


def _sc_call(dsum, hsum, idx3, wts3):
    mesh = plsc.VectorSubcoreMesh(core_axis_name="c", subcore_axis_name="s")
    f = functools.partial(
        pl.kernel,
        mesh=mesh,
        out_type=jax.ShapeDtypeStruct((_N_LEVELS, 512, 128), jnp.float32),
        scratch_types=[
            pltpu.VMEM((4, _TROWS, 128), jnp.int32),
            pltpu.VMEM((4, _TROWS, 128), jnp.float32),
            pltpu.VMEM((4, _TROWS, 128), jnp.float32),
            pltpu.VMEM((_TROWS, 128), jnp.float32),
            pltpu.VMEM((_STCH,), jnp.float32),
            pltpu.VMEM_SHARED((_SC_SLICE,), jnp.float32),
            pltpu.SemaphoreType.DMA,
        ],
    )(_sc_body)
    return f(dsum, hsum, idx3, wts3)


def kernel(xyz, batch, dense, hashtab):
    xyz = xyz.astype(jnp.float32)
    dsum = _rowsum_dense_call(dense.astype(jnp.float32))
    hsum = _rowsum_hash_call(hashtab.astype(jnp.float32))

    xs = xyz[:, 0].reshape(512, 128)
    ys = xyz[:, 1].reshape(512, 128)
    zs = xyz[:, 2].reshape(512, 128)
    idx3, wts3 = _idxw_call(xs, ys, zs)

    lv = _sc_call(dsum, hsum, idx3, wts3)
    lv = lv.reshape(_N_LEVELS, _N_POINTS).T
    return jnp.concatenate([xyz, lv], axis=-1)
